# baseline (speedup 1.0000x reference)
import functools
import math

import jax
import jax.numpy as jnp
from jax import lax
from jax.experimental import pallas as pl
from jax.experimental.pallas import tpu as pltpu
from jax.experimental.pallas import tpu_sc as plsc

NUM_CORES = 2
NUM_SUBCORES = 16
LANES = 16
NW = NUM_CORES * NUM_SUBCORES
BATCH = 16384
D = 64
WROW = 128
BPW = BATCH // NW
CHUNK = 128
NCHUNK = BPW // CHUNK
MARGIN = 1.0
TB = 1024

_mesh = plsc.VectorSubcoreMesh(
    core_axis_name="c", subcore_axis_name="s",
    num_cores=NUM_CORES, num_subcores=NUM_SUBCORES)

_GDN = jax.lax.GatherDimensionNumbers(
    offset_dims=(), collapsed_slice_dims=(0,), start_index_map=(0,))


def _lane_perm(v, idx):
    return jax.lax.gather(v, idx[:, None], _GDN, (1,),
                          mode=jax.lax.GatherScatterMode.PROMISE_IN_BOUNDS)


def _lanesum_splat(v):
    iota = lax.iota(jnp.int32, LANES)
    for shift in (8, 4, 2, 1):
        v = v + _lane_perm(v, iota ^ shift)
    return v


def _tc_transpose_body(x_ref, o_ref):
    x = x_ref[...]
    eye = jnp.eye(D, dtype=jnp.float32)
    x3 = x.reshape(D, TB // 2, 2)
    for p in range(2):
        yp = jax.lax.dot_general(
            x3[:, :, p], eye, (((0,), (0,)), ((), ())),
            preferred_element_type=jnp.float32)
        o_ref[:, p * D:(p + 1) * D] = yp


def _tc_transpose(x_t, n_rows):
    return pl.pallas_call(
        _tc_transpose_body,
        grid=(math.ceil(n_rows / TB),),
        in_specs=[pl.BlockSpec((D, TB), lambda g: (0, g))],
        out_specs=pl.BlockSpec((TB // 2, 2 * D), lambda g: (g, 0)),
        out_shape=jax.ShapeDtypeStruct((n_rows // 2, 2 * D), jnp.float32),
        compiler_params=pltpu.CompilerParams(vmem_limit_bytes=120 * 1024 * 1024),
    )(x_t)


@functools.partial(
    pl.kernel,
    mesh=_mesh,
    out_type=jax.ShapeDtypeStruct((NW * LANES,), jnp.float32),
    scratch_types=[
        pltpu.VMEM((BPW,), jnp.int32),
        pltpu.VMEM((BPW,), jnp.int32),
        pltpu.VMEM((BPW,), jnp.int32),
        pltpu.VMEM((BPW,), jnp.int32),
        pltpu.VMEM((BPW,), jnp.int32),
        pltpu.VMEM((BPW,), jnp.int32),
        pltpu.VMEM((BPW,), jnp.int32),
        pltpu.VMEM((CHUNK, WROW), jnp.float32),
        pltpu.VMEM((CHUNK, WROW), jnp.float32),
        pltpu.VMEM((CHUNK, WROW), jnp.float32),
        pltpu.VMEM((CHUNK, WROW), jnp.float32),
        pltpu.VMEM((CHUNK, WROW), jnp.float32),
        pltpu.VMEM((CHUNK, WROW), jnp.float32),
        pltpu.VMEM((LANES,), jnp.float32),
        pltpu.SemaphoreType.DMA,
    ],
)
def _sc_partials(ph_h, pr_h, pt_h, nh_h, nr_h, nt_h, pk_h, ent_h, rel_h, out_h,
                 ph_v, pr_v, pt_v, nh_v, nr_v, nt_v, pk_v,
                 hp, rp, tp, hn, rn, tn, ob, sem):
    wid = lax.axis_index("c") * NUM_SUBCORES + lax.axis_index("s")
    base = pl.multiple_of(wid * BPW, BPW)

    for src, dst in ((ph_h, ph_v), (pr_h, pr_v), (pt_h, pt_v),
                     (nh_h, nh_v), (nr_h, nr_v), (nt_h, nt_v),
                     (pk_h, pk_v)):
        pltpu.sync_copy(src.at[pl.ds(base, BPW)], dst)

    acc = jnp.zeros((LANES,), jnp.float32)
    for k in range(NCHUNK):
        sl = pl.ds(k * CHUNK, CHUNK)
        cps = [
            pltpu.async_copy(ent_h.at[ph_v.at[sl]], hp, sem),
            pltpu.async_copy(rel_h.at[pr_v.at[sl]], rp, sem),
            pltpu.async_copy(ent_h.at[pt_v.at[sl]], tp, sem),
            pltpu.async_copy(ent_h.at[nh_v.at[sl]], hn, sem),
            pltpu.async_copy(rel_h.at[nr_v.at[sl]], rn, sem),
            pltpu.async_copy(ent_h.at[nt_v.at[sl]], tn, sem),
        ]
        for cp in cps:
            cp.wait()

        def group(g, a):
            pk_vec = pk_v[pl.ds(pl.multiple_of(k * CHUNK + g * LANES, LANES),
                                LANES)]
            gbase = g * LANES
            for l in range(LANES):
                pk = pk_vec[l]
                i = gbase + l
                o_ph = pl.multiple_of((pk & 1) * D, D)
                o_pr = pl.multiple_of(((pk >> 1) & 1) * D, D)
                o_pt = pl.multiple_of(((pk >> 2) & 1) * D, D)
                o_nh = pl.multiple_of(((pk >> 3) & 1) * D, D)
                o_nr = pl.multiple_of(((pk >> 4) & 1) * D, D)
                o_nt = pl.multiple_of(((pk >> 5) & 1) * D, D)
                dsum = None
                for j in range(D // LANES):
                    jo = j * LANES
                    vp = jnp.abs(hp[i, pl.ds(o_ph + jo, LANES)]
                                 + rp[i, pl.ds(o_pr + jo, LANES)]
                                 - tp[i, pl.ds(o_pt + jo, LANES)])
                    vn = jnp.abs(hn[i, pl.ds(o_nh + jo, LANES)]
                                 + rn[i, pl.ds(o_nr + jo, LANES)]
                                 - tn[i, pl.ds(o_nt + jo, LANES)])
                    dj = vp - vn
                    dsum = dj if dsum is None else dsum + dj
                diff = _lanesum_splat(dsum)
                a = a + jnp.maximum(diff + MARGIN, 0.0)
            return a

        acc = lax.fori_loop(0, CHUNK // LANES, group, acc)

    ob[...] = acc
    pltpu.sync_copy(ob, out_h.at[pl.ds(pl.multiple_of(wid * LANES, LANES), LANES)])


def _tc_reduce(x_ref, o_ref):
    o_ref[...] = jnp.full((1, 1), jnp.sum(x_ref[...]) * (1.0 / (LANES * BATCH)),
                          jnp.float32)


def kernel(pos_triples, neg_triples, entity_emb, relation_emb):
    cols = [pos_triples[:, 0], pos_triples[:, 1], pos_triples[:, 2],
            neg_triples[:, 0], neg_triples[:, 1], neg_triples[:, 2]]
    cols = [c.astype(jnp.int32) for c in cols]
    rows = [c >> 1 for c in cols]
    pk = (cols[0] & 1)
    for b, c in enumerate(cols[1:], start=1):
        pk = pk | ((c & 1) << b)
    ent2 = _tc_transpose(entity_emb.T, entity_emb.shape[0])
    rel2 = _tc_transpose(relation_emb.T, relation_emb.shape[0])
    partials = _sc_partials(*rows, pk, ent2, rel2)
    loss = pl.pallas_call(
        _tc_reduce,
        out_shape=jax.ShapeDtypeStruct((1, 1), jnp.float32),
    )(partials.reshape(4, NW * LANES // 4))
    return loss[0, 0]


# speedup vs baseline: 17.8412x; 17.8412x over previous
import functools
import math

import jax
import jax.numpy as jnp
from jax import lax
from jax.experimental import pallas as pl
from jax.experimental.pallas import tpu as pltpu
from jax.experimental.pallas import tpu_sc as plsc

NUM_CORES = 2
NUM_SUBCORES = 16
LANES = 16
NW = NUM_CORES * NUM_SUBCORES
BATCH = 16384
D = 64
WROW = 128
BPW = BATCH // NW
CHUNK = 128
NCHUNK = BPW // CHUNK
MARGIN = 1.0
TB = 512
SPLIT = 500224
NROWS2 = SPLIT

_mesh = plsc.VectorSubcoreMesh(
    core_axis_name="c", subcore_axis_name="s",
    num_cores=NUM_CORES, num_subcores=NUM_SUBCORES)

_GDN = jax.lax.GatherDimensionNumbers(
    offset_dims=(), collapsed_slice_dims=(0,), start_index_map=(0,))


def _lane_perm(v, idx):
    return jax.lax.gather(v, idx[:, None], _GDN, (1,),
                          mode=jax.lax.GatherScatterMode.PROMISE_IN_BOUNDS)


def _lanesum_splat(v):
    iota = lax.iota(jnp.int32, LANES)
    for shift in (8, 4, 2, 1):
        v = v + _lane_perm(v, iota ^ shift)
    return v


def _tc_transpose_body(xa_ref, xb_ref, o_ref):
    eye = jnp.eye(D, dtype=jnp.float32)
    ya = jax.lax.dot_general(
        xa_ref[...], eye, (((0,), (0,)), ((), ())),
        preferred_element_type=jnp.float32)
    yb = jax.lax.dot_general(
        xb_ref[...], eye, (((0,), (0,)), ((), ())),
        preferred_element_type=jnp.float32)
    o_ref[:, 0:D] = ya
    o_ref[:, D:2 * D] = yb


def _tc_transpose(x_t):
    return pl.pallas_call(
        _tc_transpose_body,
        grid=(SPLIT // TB,),
        in_specs=[pl.BlockSpec((D, TB), lambda g: (0, g)),
                  pl.BlockSpec((D, TB), lambda g: (0, SPLIT // TB + g))],
        out_specs=pl.BlockSpec((TB, 2 * D), lambda g: (g, 0)),
        out_shape=jax.ShapeDtypeStruct((NROWS2, 2 * D), jnp.float32),
        compiler_params=pltpu.CompilerParams(vmem_limit_bytes=120 * 1024 * 1024),
    )(x_t, x_t)


@functools.partial(
    pl.kernel,
    mesh=_mesh,
    out_type=jax.ShapeDtypeStruct((NW * LANES,), jnp.float32),
    scratch_types=[
        pltpu.VMEM((BPW,), jnp.int32),
        pltpu.VMEM((BPW,), jnp.int32),
        pltpu.VMEM((BPW,), jnp.int32),
        pltpu.VMEM((BPW,), jnp.int32),
        pltpu.VMEM((BPW,), jnp.int32),
        pltpu.VMEM((BPW,), jnp.int32),
        pltpu.VMEM((BPW,), jnp.int32),
        pltpu.VMEM((CHUNK, WROW), jnp.float32),
        pltpu.VMEM((CHUNK, WROW), jnp.float32),
        pltpu.VMEM((CHUNK, WROW), jnp.float32),
        pltpu.VMEM((CHUNK, WROW), jnp.float32),
        pltpu.VMEM((CHUNK, WROW), jnp.float32),
        pltpu.VMEM((CHUNK, WROW), jnp.float32),
        pltpu.VMEM((LANES,), jnp.float32),
        pltpu.SemaphoreType.DMA,
    ],
)
def _sc_partials(ph_h, pr_h, pt_h, nh_h, nr_h, nt_h, pk_h, ent_h, rel_h, out_h,
                 ph_v, pr_v, pt_v, nh_v, nr_v, nt_v, pk_v,
                 hp, rp, tp, hn, rn, tn, ob, sem):
    wid = lax.axis_index("c") * NUM_SUBCORES + lax.axis_index("s")
    base = pl.multiple_of(wid * BPW, BPW)

    for src, dst in ((ph_h, ph_v), (pr_h, pr_v), (pt_h, pt_v),
                     (nh_h, nh_v), (nr_h, nr_v), (nt_h, nt_v),
                     (pk_h, pk_v)):
        pltpu.sync_copy(src.at[pl.ds(base, BPW)], dst)

    acc = jnp.zeros((LANES,), jnp.float32)
    for k in range(NCHUNK):
        sl = pl.ds(k * CHUNK, CHUNK)
        cps = [
            pltpu.async_copy(ent_h.at[ph_v.at[sl]], hp, sem),
            pltpu.async_copy(rel_h.at[pr_v.at[sl]], rp, sem),
            pltpu.async_copy(ent_h.at[pt_v.at[sl]], tp, sem),
            pltpu.async_copy(ent_h.at[nh_v.at[sl]], hn, sem),
            pltpu.async_copy(rel_h.at[nr_v.at[sl]], rn, sem),
            pltpu.async_copy(ent_h.at[nt_v.at[sl]], tn, sem),
        ]
        for cp in cps:
            cp.wait()

        def group(g, a):
            pk_vec = pk_v[pl.ds(pl.multiple_of(k * CHUNK + g * LANES, LANES),
                                LANES)]
            gbase = g * LANES
            for l in range(LANES):
                pk = pk_vec[l]
                i = gbase + l
                o_ph = pl.multiple_of((pk & 1) * D, D)
                o_pr = pl.multiple_of(((pk >> 1) & 1) * D, D)
                o_pt = pl.multiple_of(((pk >> 2) & 1) * D, D)
                o_nh = pl.multiple_of(((pk >> 3) & 1) * D, D)
                o_nr = pl.multiple_of(((pk >> 4) & 1) * D, D)
                o_nt = pl.multiple_of(((pk >> 5) & 1) * D, D)
                dsum = None
                for j in range(D // LANES):
                    jo = j * LANES
                    vp = jnp.abs(hp[i, pl.ds(o_ph + jo, LANES)]
                                 + rp[i, pl.ds(o_pr + jo, LANES)]
                                 - tp[i, pl.ds(o_pt + jo, LANES)])
                    vn = jnp.abs(hn[i, pl.ds(o_nh + jo, LANES)]
                                 + rn[i, pl.ds(o_nr + jo, LANES)]
                                 - tn[i, pl.ds(o_nt + jo, LANES)])
                    dj = vp - vn
                    dsum = dj if dsum is None else dsum + dj
                diff = _lanesum_splat(dsum)
                a = a + jnp.maximum(diff + MARGIN, 0.0)
            return a

        acc = lax.fori_loop(0, CHUNK // LANES, group, acc)

    ob[...] = acc
    pltpu.sync_copy(ob, out_h.at[pl.ds(pl.multiple_of(wid * LANES, LANES), LANES)])


def _tc_reduce(x_ref, o_ref):
    o_ref[...] = jnp.full((1, 1), jnp.sum(x_ref[...]) * (1.0 / (LANES * BATCH)),
                          jnp.float32)


def kernel(pos_triples, neg_triples, entity_emb, relation_emb):
    cols = [pos_triples[:, 0], pos_triples[:, 1], pos_triples[:, 2],
            neg_triples[:, 0], neg_triples[:, 1], neg_triples[:, 2]]
    cols = [c.astype(jnp.int32) for c in cols]
    pars = [(c >= SPLIT).astype(jnp.int32) for c in cols]
    rows = [c - p * SPLIT for c, p in zip(cols, pars)]
    pk = pars[0]
    for b, p in enumerate(pars[1:], start=1):
        pk = pk | (p << b)
    ent2 = _tc_transpose(entity_emb.T)
    rel2 = _tc_transpose(relation_emb.T)
    partials = _sc_partials(*rows, pk, ent2, rel2)
    loss = pl.pallas_call(
        _tc_reduce,
        out_shape=jax.ShapeDtypeStruct((1, 1), jnp.float32),
    )(partials.reshape(4, NW * LANES // 4))
    return loss[0, 0]


# speedup vs baseline: 35.8528x; 2.0096x over previous
import functools

import jax
import jax.numpy as jnp
from jax import lax
from jax.experimental import pallas as pl
from jax.experimental.pallas import tpu as pltpu
from jax.experimental.pallas import tpu_sc as plsc

NUM_CORES = 2
NUM_SUBCORES = 16
LANES = 16
NW = NUM_CORES * NUM_SUBCORES
BATCH = 16384
D = 64
BPW = BATCH // NW
NGROUP = BPW // LANES
MARGIN = 1.0

_mesh = plsc.VectorSubcoreMesh(
    core_axis_name="c", subcore_axis_name="s",
    num_cores=NUM_CORES, num_subcores=NUM_SUBCORES)

_GDN = jax.lax.GatherDimensionNumbers(
    offset_dims=(), collapsed_slice_dims=(0,), start_index_map=(0,))


def _lane_perm(v, idx):
    return jax.lax.gather(v, idx[:, None], _GDN, (1,),
                          mode=jax.lax.GatherScatterMode.PROMISE_IN_BOUNDS)


def _lanesum_splat(v):
    iota = lax.iota(jnp.int32, LANES)
    for shift in (8, 4, 2, 1):
        v = v + _lane_perm(v, iota ^ shift)
    return v


@functools.partial(
    pl.kernel,
    mesh=_mesh,
    out_type=jax.ShapeDtypeStruct((NW * LANES,), jnp.float32),
    scratch_types=[
        pltpu.VMEM((BPW,), jnp.int32),
        pltpu.VMEM((BPW,), jnp.int32),
        pltpu.VMEM((BPW,), jnp.int32),
        pltpu.VMEM((BPW,), jnp.int32),
        pltpu.VMEM((BPW,), jnp.int32),
        pltpu.VMEM((BPW,), jnp.int32),
        pltpu.VMEM((LANES, D), jnp.float32),
        pltpu.VMEM((LANES, D), jnp.float32),
        pltpu.VMEM((LANES, D), jnp.float32),
        pltpu.VMEM((LANES, D), jnp.float32),
        pltpu.VMEM((LANES, D), jnp.float32),
        pltpu.VMEM((LANES, D), jnp.float32),
        pltpu.VMEM((LANES,), jnp.float32),
        pltpu.SemaphoreType.DMA,
    ],
)
def _sc_partials(ph_h, pr_h, pt_h, nh_h, nr_h, nt_h, ent_h, rel_h, out_h,
                 ph_v, pr_v, pt_v, nh_v, nr_v, nt_v,
                 hp, rp, tp, hn, rn, tn, ob, sem):
    wid = lax.axis_index("c") * NUM_SUBCORES + lax.axis_index("s")
    base = pl.multiple_of(wid * BPW, BPW)

    for src, dst in ((ph_h, ph_v), (pr_h, pr_v), (pt_h, pt_v),
                     (nh_h, nh_v), (nr_h, nr_v), (nt_h, nt_v)):
        pltpu.sync_copy(src.at[pl.ds(base, BPW)], dst)

    roles = ((ph_v, ent_h, hp), (pr_v, rel_h, rp), (pt_v, ent_h, tp),
             (nh_v, ent_h, hn), (nr_v, rel_h, rn), (nt_v, ent_h, tn))

    def group(g, acc):
        gsl = pl.ds(pl.multiple_of(g * LANES, LANES), LANES)
        cps = []
        for idx_v, tbl_h, buf in roles:
            iv = idx_v[gsl]
            for l in range(LANES):
                cps.append(pltpu.async_copy(tbl_h.at[iv[l]], buf.at[l], sem))
        for cp in cps:
            cp.wait()

        for l in range(LANES):
            dsum = None
            for j in range(D // LANES):
                js = pl.ds(j * LANES, LANES)
                vp = jnp.abs(hp[l, js] + rp[l, js] - tp[l, js])
                vn = jnp.abs(hn[l, js] + rn[l, js] - tn[l, js])
                dj = vp - vn
                dsum = dj if dsum is None else dsum + dj
            diff = _lanesum_splat(dsum)
            acc = acc + jnp.maximum(diff + MARGIN, 0.0)
        return acc

    acc = lax.fori_loop(0, NGROUP, group, jnp.zeros((LANES,), jnp.float32))

    ob[...] = acc
    pltpu.sync_copy(ob, out_h.at[pl.ds(pl.multiple_of(wid * LANES, LANES), LANES)])


def _tc_reduce(x_ref, o_ref):
    o_ref[...] = jnp.full((1, 1), jnp.sum(x_ref[...]) * (1.0 / (LANES * BATCH)),
                          jnp.float32)


def kernel(pos_triples, neg_triples, entity_emb, relation_emb):
    cols = [pos_triples[:, 0], pos_triples[:, 1], pos_triples[:, 2],
            neg_triples[:, 0], neg_triples[:, 1], neg_triples[:, 2]]
    cols = [c.astype(jnp.int32) for c in cols]
    partials = _sc_partials(*cols, entity_emb, relation_emb)
    loss = pl.pallas_call(
        _tc_reduce,
        out_shape=jax.ShapeDtypeStruct((1, 1), jnp.float32),
    )(partials.reshape(4, NW * LANES // 4))
    return loss[0, 0]


# speedup vs baseline: 53.3756x; 1.4887x over previous
import functools

import jax
import jax.numpy as jnp
from jax import lax
from jax.experimental import pallas as pl
from jax.experimental.pallas import tpu as pltpu
from jax.experimental.pallas import tpu_sc as plsc

NUM_CORES = 2
NUM_SUBCORES = 16
LANES = 16
NW = NUM_CORES * NUM_SUBCORES
BATCH = 16384
D = 64
BPW = BATCH // NW
NGROUP = BPW // LANES
MARGIN = 1.0

_mesh = plsc.VectorSubcoreMesh(
    core_axis_name="c", subcore_axis_name="s",
    num_cores=NUM_CORES, num_subcores=NUM_SUBCORES)

_GDN = jax.lax.GatherDimensionNumbers(
    offset_dims=(), collapsed_slice_dims=(0,), start_index_map=(0,))


def _lane_perm(v, idx):
    return jax.lax.gather(v, idx[:, None], _GDN, (1,),
                          mode=jax.lax.GatherScatterMode.PROMISE_IN_BOUNDS)


def _lanesum_splat(v):
    iota = lax.iota(jnp.int32, LANES)
    for shift in (8, 4, 2, 1):
        v = v + _lane_perm(v, iota ^ shift)
    return v


@functools.partial(
    pl.kernel,
    mesh=_mesh,
    out_type=jax.ShapeDtypeStruct((NW * LANES,), jnp.float32),
    scratch_types=[
        pltpu.VMEM((BPW,), jnp.int32),
        pltpu.VMEM((BPW,), jnp.int32),
        pltpu.VMEM((BPW,), jnp.int32),
        pltpu.VMEM((BPW,), jnp.int32),
        pltpu.VMEM((BPW,), jnp.int32),
        pltpu.VMEM((BPW,), jnp.int32),
        pltpu.VMEM((LANES, D), jnp.float32),
        pltpu.VMEM((LANES, D), jnp.float32),
        pltpu.VMEM((LANES, D), jnp.float32),
        pltpu.VMEM((LANES, D), jnp.float32),
        pltpu.VMEM((LANES, D), jnp.float32),
        pltpu.VMEM((LANES, D), jnp.float32),
        pltpu.VMEM((LANES,), jnp.float32),
        pltpu.SemaphoreType.DMA,
    ],
)
def _sc_partials(ph_h, pr_h, pt_h, nh_h, nr_h, nt_h, ent_h, rel_h, out_h,
                 ph_v, pr_v, pt_v, nh_v, nr_v, nt_v,
                 hp, rp, tp, hn, rn, tn, ob, sem):
    wid = lax.axis_index("c") * NUM_SUBCORES + lax.axis_index("s")
    base = pl.multiple_of(wid * BPW, BPW)

    for src, dst in ((ph_h, ph_v), (pr_h, pr_v), (pt_h, pt_v),
                     (nh_h, nh_v), (nr_h, nr_v), (nt_h, nt_v)):
        pltpu.sync_copy(src.at[pl.ds(base, BPW)], dst)

    roles = ((ph_v, ent_h, hp), (pr_v, rel_h, rp), (pt_v, ent_h, tp),
             (nh_v, ent_h, hn), (nr_v, rel_h, rn), (nt_v, ent_h, tn))

    def group(g, acc):
        gsl = pl.ds(pl.multiple_of(g * LANES, LANES), LANES)
        cps = []
        for idx_v, tbl_h, buf in roles:
            iv = idx_v[gsl]
            for l in range(LANES):
                cps.append(pltpu.async_copy(
                    tbl_h.at[iv[l] >> 3, iv[l] & 7], buf.at[l], sem))
        for cp in cps:
            cp.wait()

        for l in range(LANES):
            dsum = None
            for j in range(D // LANES):
                js = pl.ds(j * LANES, LANES)
                vp = jnp.abs(hp[l, js] + rp[l, js] - tp[l, js])
                vn = jnp.abs(hn[l, js] + rn[l, js] - tn[l, js])
                dj = vp - vn
                dsum = dj if dsum is None else dsum + dj
            diff = _lanesum_splat(dsum)
            acc = acc + jnp.maximum(diff + MARGIN, 0.0)
        return acc

    acc = lax.fori_loop(0, NGROUP, group, jnp.zeros((LANES,), jnp.float32))

    ob[...] = acc
    pltpu.sync_copy(ob, out_h.at[pl.ds(pl.multiple_of(wid * LANES, LANES), LANES)])


def _tc_reduce(x_ref, o_ref):
    o_ref[...] = jnp.full((1, 1), jnp.sum(x_ref[...]) * (1.0 / (LANES * BATCH)),
                          jnp.float32)


def kernel(pos_triples, neg_triples, entity_emb, relation_emb):
    cols = [pos_triples[:, 0], pos_triples[:, 1], pos_triples[:, 2],
            neg_triples[:, 0], neg_triples[:, 1], neg_triples[:, 2]]
    cols = [c.astype(jnp.int32) for c in cols]
    ent3 = entity_emb.reshape(entity_emb.shape[0] // 8, 8, D)
    rel3 = relation_emb.reshape(relation_emb.shape[0] // 8, 8, D)
    partials = _sc_partials(*cols, ent3, rel3)
    loss = pl.pallas_call(
        _tc_reduce,
        out_shape=jax.ShapeDtypeStruct((1, 1), jnp.float32),
    )(partials.reshape(4, NW * LANES // 4))
    return loss[0, 0]
